# Initial kernel scaffold; baseline (speedup 1.0000x reference)
import jax
import jax.numpy as jnp
from jax.experimental import pallas as pl


def kernel(x, table):
    raise NotImplementedError("write your pallas kernel here")


# speedup vs baseline: 1.5478x; 1.5478x over previous
import functools

import jax
import jax.numpy as jnp
from jax import lax
from jax.experimental import pallas as pl
from jax.experimental.pallas import tpu as pltpu
from jax.experimental.pallas import tpu_sc as plsc

_NC = 2
_NS = 16
_NW = _NC * _NS


def _make_gather(B, V, D, chunk):
    nchunk = B // (_NW * chunk)
    assert B == nchunk * _NW * chunk
    mesh = plsc.VectorSubcoreMesh(core_axis_name="c", subcore_axis_name="s")

    @functools.partial(
        pl.kernel,
        mesh=mesh,
        out_type=jax.ShapeDtypeStruct((B, D), jnp.float32),
        scratch_types=[
            pltpu.VMEM((chunk,), jnp.int32),
            pltpu.VMEM((chunk, D), jnp.float32),
            pltpu.SemaphoreType.DMA,
        ],
        compiler_params=pltpu.CompilerParams(use_tc_tiling_on_sc=False),
    )
    def gather_kernel(table_hbm, idx_hbm, out_hbm, idx_v, rows_v, sem):
        wid = lax.axis_index("s") * _NC + lax.axis_index("c")
        base = wid * (nchunk * chunk)

        def body(i, carry):
            off = base + i * chunk
            pltpu.sync_copy(idx_hbm.at[pl.ds(off, chunk)], idx_v)
            pltpu.async_copy(table_hbm.at[idx_v], rows_v, sem).wait()
            pltpu.sync_copy(rows_v, out_hbm.at[pl.ds(off, chunk)])
            return carry

        lax.fori_loop(0, nchunk, body, 0)

    return gather_kernel


def kernel(x, table):
    orig_shape = x.shape
    B = x.size
    V, D = table.shape
    xf = x.reshape(B).astype(jnp.int32)
    out = _make_gather(B, V, D, 1024)(table, xf)
    return out.reshape(*orig_shape, D)


# speedup vs baseline: 1.5732x; 1.0164x over previous
import functools

import jax
import jax.numpy as jnp
from jax import lax
from jax.experimental import pallas as pl
from jax.experimental.pallas import tpu as pltpu
from jax.experimental.pallas import tpu_sc as plsc

_NC = 2
_NS = 16
_NW = _NC * _NS


def _make_gather(B, V, D, chunk, nbuf):
    nchunk = B // (_NW * chunk)
    assert B == nchunk * _NW * chunk
    per_w = nchunk * chunk
    mesh = plsc.VectorSubcoreMesh(core_axis_name="c", subcore_axis_name="s")

    @functools.partial(
        pl.kernel,
        mesh=mesh,
        out_type=jax.ShapeDtypeStruct((B, D), jnp.float32),
        scratch_types=[
            pltpu.VMEM((per_w,), jnp.int32),
            *[pltpu.VMEM((chunk, D), jnp.float32) for _ in range(nbuf)],
            pltpu.SemaphoreType.DMA((nbuf,)),
            pltpu.SemaphoreType.DMA((nbuf,)),
        ],
        compiler_params=pltpu.CompilerParams(use_tc_tiling_on_sc=False),
    )
    def gather_kernel(table_hbm, idx_hbm, out_hbm, idx_v, *rest):
        rows = rest[:nbuf]
        gsem, ssem = rest[nbuf], rest[nbuf + 1]
        wid = lax.axis_index("s") * _NC + lax.axis_index("c")
        base = wid * per_w

        pltpu.sync_copy(idx_hbm.at[pl.ds(base, per_w)], idx_v)

        gathers = [None] * nchunk
        stores = [None] * nchunk
        store_waited = [False] * nchunk

        def start_gather(i):
            b = i % nbuf
            gathers[i] = pltpu.async_copy(
                table_hbm.at[idx_v.at[pl.ds(i * chunk, chunk)]],
                rows[b],
                gsem.at[b],
            )

        for i in range(min(nbuf, nchunk)):
            start_gather(i)

        for i in range(nchunk):
            b = i % nbuf
            gathers[i].wait()
            stores[i] = pltpu.async_copy(
                rows[b], out_hbm.at[pl.ds(base + i * chunk, chunk)], ssem.at[b]
            )
            j = i - 1
            if j >= 0 and j + nbuf < nchunk:
                stores[j].wait()
                store_waited[j] = True
                start_gather(j + nbuf)

        for i in range(nchunk):
            if not store_waited[i]:
                stores[i].wait()

    return gather_kernel


def kernel(x, table):
    orig_shape = x.shape
    B = x.size
    V, D = table.shape
    xf = x.reshape(B).astype(jnp.int32)
    out = _make_gather(B, V, D, 1024, 3)(table, xf)
    return out.reshape(*orig_shape, D)
